# baseline (speedup 1.0000x reference)
import functools

import jax
import jax.numpy as jnp
from jax import lax
from jax.experimental import pallas as pl
from jax.experimental.pallas import tpu as pltpu
from jax.experimental.pallas import tpu_sc as plsc

N = 10000
NPAD = 10240
NCLS = 40
D = 64
NC, NS, L = 2, 16, 16
NW = NC * NS
CH = 128
RB = 512
NBLK = NPAD // RB

_mesh = functools.partial(
    plsc.VectorSubcoreMesh, core_axis_name="c", subcore_axis_name="s",
    num_cores=NC, num_subcores=NS)

_sc_params = pltpu.CompilerParams(needs_layout_passes=False,
                                  use_tc_tiling_on_sc=False)


def _make_sc_pass1(epw):
    cpg = epw // 16

    @functools.partial(
        pl.kernel,
        out_type=(
            jax.ShapeDtypeStruct((NW, NPAD), jnp.float32),
            jax.ShapeDtypeStruct((NW, NPAD), jnp.float32),
            jax.ShapeDtypeStruct((NW * epw,), jnp.int32),
        ),
        mesh=_mesh(),
        compiler_params=_sc_params,
        scratch_types=[
            pltpu.VMEM((epw,), jnp.int32),
            pltpu.VMEM((epw,), jnp.int32),
            pltpu.VMEM((epw,), jnp.float32),
            pltpu.VMEM((epw,), jnp.int32),
            pltpu.VMEM((NPAD,), jnp.float32),
            pltpu.VMEM((NPAD,), jnp.float32),
        ],
    )
    def sc_pass1(rowh, colh, eah, z1h, degp, cntp, colrh,
                 rv, cv, av, crv, dacc, cacc):
        wid = lax.axis_index("s") * NC + lax.axis_index("c")
        base = wid * epw
        pltpu.sync_copy(rowh.at[pl.ds(base, epw)], rv)
        pltpu.sync_copy(colh.at[pl.ds(base, epw)], cv)
        pltpu.sync_copy(eah.at[pl.ds(base, epw)], av)
        pltpu.sync_copy(z1h, dacc)
        pltpu.sync_copy(z1h, cacc)

        def body(g, carry):
            sl = pl.ds(g * 16, 16)
            r = rv[sl]
            c = cv[sl]
            a = av[sl]
            k = r != c
            plsc.addupdate_scatter(dacc, [c], jnp.where(k, jnp.abs(a), 0.0))
            plsc.addupdate_scatter(cacc, [c], jnp.where(k, 1.0, 0.0))
            crv[sl] = jnp.where(k, c, N)
            return carry

        lax.fori_loop(0, cpg, body, 0)
        pltpu.sync_copy(dacc, degp.at[wid])
        pltpu.sync_copy(cacc, cntp.at[wid])
        pltpu.sync_copy(crv, colrh.at[pl.ds(base, epw)])

    return sc_pass1


def _make_sc_pass2(epw):
    cpg = epw // 16

    @functools.partial(
        pl.kernel,
        out_type=jax.ShapeDtypeStruct((NW, NPAD), jnp.float32),
        mesh=_mesh(),
        compiler_params=_sc_params,
        scratch_types=[
            pltpu.VMEM((epw,), jnp.int32),
            pltpu.VMEM((epw,), jnp.int32),
            pltpu.VMEM((epw,), jnp.float32),
            pltpu.VMEM((NPAD,), jnp.float32),
            pltpu.VMEM((NPAD,), jnp.float32),
        ],
    )
    def sc_pass2(rowh, colh, eah, dinvh, z1h, partp, rv, cv, av, dv, pacc):
        wid = lax.axis_index("s") * NC + lax.axis_index("c")
        base = wid * epw
        pltpu.sync_copy(rowh.at[pl.ds(base, epw)], rv)
        pltpu.sync_copy(colh.at[pl.ds(base, epw)], cv)
        pltpu.sync_copy(eah.at[pl.ds(base, epw)], av)
        pltpu.sync_copy(dinvh, dv)
        pltpu.sync_copy(z1h, pacc)

        def body(g, carry):
            sl = pl.ds(g * 16, 16)
            r = rv[sl]
            c = cv[sl]
            a = av[sl]
            k = r != c
            dr = plsc.load_gather(dv, [r])
            plsc.addupdate_scatter(pacc, [c],
                                   jnp.where(k, jnp.abs(a), 0.0) * dr)
            return carry

        lax.fori_loop(0, cpg, body, 0)
        pltpu.sync_copy(pacc, partp.at[wid])

    return sc_pass2


def _make_sc_pass2_agg(epw):
    cpg = epw // 16
    cpw = epw // CH
    rows_per_sub = NPAD // NS

    @functools.partial(
        pl.kernel,
        out_type=(
            jax.ShapeDtypeStruct((NW, NPAD), jnp.float32),
            jax.ShapeDtypeStruct((NC, NPAD, D), jnp.bfloat16),
        ),
        mesh=_mesh(),
        compiler_params=_sc_params,
        scratch_types=[
            pltpu.VMEM((epw,), jnp.int32),
            pltpu.VMEM((epw,), jnp.int32),
            pltpu.VMEM((epw,), jnp.float32),
            pltpu.VMEM((NPAD,), jnp.float32),
            pltpu.VMEM((NPAD,), jnp.float32),
            pltpu.VMEM((cpw, CH), jnp.int32),
            pltpu.VMEM((CH, D), jnp.bfloat16),
            pltpu.VMEM((CH, D), jnp.bfloat16),
            pltpu.VMEM_SHARED((NPAD, D), jnp.bfloat16),
            pltpu.SemaphoreType.DMA,
            pltpu.SemaphoreType.DMA,
        ],
    )
    def sc_pass2_agg(rowh, colh, eah, dinvh, z1h, hh, colh2, z2h,
                     partp, aggout,
                     rv, cv, av, dv, pacc, cv2, buf0, buf1, acc,
                     sg0, sg1):
        c = lax.axis_index("c")
        s = lax.axis_index("s")
        wid = s * NC + c
        base = wid * epw

        pltpu.sync_copy(rowh.at[pl.ds(base, epw)], rv)
        pltpu.sync_copy(colh.at[pl.ds(base, epw)], cv)
        pltpu.sync_copy(eah.at[pl.ds(base, epw)], av)
        pltpu.sync_copy(dinvh, dv)
        pltpu.sync_copy(z1h, pacc)

        def body(g, carry):
            sl = pl.ds(g * 16, 16)
            r = rv[sl]
            cc = cv[sl]
            a = av[sl]
            k = r != cc
            dr = plsc.load_gather(dv, [r])
            plsc.addupdate_scatter(pacc, [cc],
                                   jnp.where(k, jnp.abs(a), 0.0) * dr)
            return carry

        lax.fori_loop(0, cpg, body, 0)
        pltpu.sync_copy(pacc, partp.at[wid])

        pltpu.sync_copy(z2h, acc.at[pl.ds(s * rows_per_sub, rows_per_sub)])
        pltpu.sync_copy(colh2.at[pl.ds(wid * cpw, cpw)], cv2)
        plsc.subcore_barrier()

        def ridx(j):
            return rv.at[pl.ds(j * CH, CH)]

        pltpu.async_copy(hh.at[ridx(0)], buf0, sg0)

        def abody(jj, carry):
            j0 = 2 * jj
            pltpu.async_copy(hh.at[ridx(j0 + 1)], buf1, sg1)
            pltpu.make_async_copy(hh.at[ridx(j0)], buf0, sg0).wait()
            pltpu.sync_copy(buf0, acc.at[cv2.at[j0]], add=True)

            @pl.when(j0 + 2 < cpw)
            def _():
                pltpu.async_copy(hh.at[ridx(j0 + 2)], buf0, sg0)

            pltpu.make_async_copy(hh.at[ridx(j0 + 1)], buf1, sg1).wait()
            pltpu.sync_copy(buf1, acc.at[cv2.at[j0 + 1]], add=True)
            return carry

        lax.fori_loop(0, cpw // 2, abody, 0)
        plsc.subcore_barrier()
        sl = pl.ds(s * rows_per_sub, rows_per_sub)
        pltpu.sync_copy(acc.at[sl], aggout.at[c, sl])

    return sc_pass2_agg


def _make_sc_agg(cpw):
    rows_per_sub = NPAD // NS

    @functools.partial(
        pl.kernel,
        out_type=jax.ShapeDtypeStruct((NC, NPAD, D), jnp.bfloat16),
        mesh=_mesh(),
        compiler_params=_sc_params,
        scratch_types=[
            pltpu.VMEM((cpw, CH), jnp.int32),
            pltpu.VMEM((cpw, CH), jnp.int32),
            pltpu.VMEM((CH, D), jnp.bfloat16),
            pltpu.VMEM((CH, D), jnp.bfloat16),
            pltpu.VMEM_SHARED((NPAD, D), jnp.bfloat16),
            pltpu.SemaphoreType.DMA,
            pltpu.SemaphoreType.DMA,
        ],
    )
    def sc_agg(hh, rowh2, colh2, z2h, aggout, rv2, cv2, buf0, buf1, acc,
               sg0, sg1):
        c = lax.axis_index("c")
        s = lax.axis_index("s")
        wid = s * NC + c
        pltpu.sync_copy(z2h, acc.at[pl.ds(s * rows_per_sub, rows_per_sub)])
        pltpu.sync_copy(rowh2.at[pl.ds(wid * cpw, cpw)], rv2)
        pltpu.sync_copy(colh2.at[pl.ds(wid * cpw, cpw)], cv2)
        plsc.subcore_barrier()

        pltpu.async_copy(hh.at[rv2.at[0]], buf0, sg0)

        def body(jj, carry):
            j0 = 2 * jj
            pltpu.async_copy(hh.at[rv2.at[j0 + 1]], buf1, sg1)
            pltpu.make_async_copy(hh.at[rv2.at[j0]], buf0, sg0).wait()
            pltpu.sync_copy(buf0, acc.at[cv2.at[j0]], add=True)

            @pl.when(j0 + 2 < cpw)
            def _():
                pltpu.async_copy(hh.at[rv2.at[j0 + 2]], buf0, sg0)

            pltpu.make_async_copy(hh.at[rv2.at[j0 + 1]], buf1, sg1).wait()
            pltpu.sync_copy(buf1, acc.at[cv2.at[j0 + 1]], add=True)
            return carry

        lax.fori_loop(0, cpw // 2, body, 0)
        plsc.subcore_barrier()
        sl = pl.ds(s * rows_per_sub, rows_per_sub)
        pltpu.sync_copy(acc.at[sl], aggout.at[c, sl])

    return sc_agg


def _t1_body(degp_ref, cntp_ref, dinv_ref, cnt_ref):
    dsum = jnp.sum(degp_ref[...], axis=0) + 1.0
    csum = jnp.sum(cntp_ref[...], axis=0) + 1.0
    dinv_ref[...] = lax.rsqrt(dsum)[:, None]
    cnt_ref[...] = csum[:, None]


def _t2_body(partp_ref, dinv_ref, sew_ref):
    psum = jnp.sum(partp_ref[...], axis=0)[:, None]
    dv = dinv_ref[...]
    sew_ref[...] = dv * (psum + dv)


def _mm_body(z_ref, w_ref, h_ref, hb_ref):
    h = jnp.dot(z_ref[...], w_ref[...], preferred_element_type=jnp.float32)
    h_ref[...] = h
    hb_ref[...] = h.astype(jnp.bfloat16)


def _comb_body(h_ref, agg_ref, cnt_ref, sew_ref, a_ref, b_ref, w_ref,
               lb_ref, bb_ref, wn_ref, o_ref, ob_ref):
    i = pl.program_id(0)
    h = h_ref[...]
    at = (agg_ref[0] + agg_ref[1]).astype(jnp.float32) + h
    t = jnp.dot(h, a_ref[...], preferred_element_type=jnp.float32) + lb_ref[...]
    u = jnp.dot(at, b_ref[...], preferred_element_type=jnp.float32)
    o = cnt_ref[...] * t + u + sew_ref[...] * w_ref[...] + bb_ref[...]
    rows = i * RB + lax.broadcasted_iota(jnp.int32, (RB, 1), 0)
    o = jnp.maximum(jnp.where(rows < N, o, 0.0), 0.0)
    hn = jnp.dot(o, wn_ref[...], preferred_element_type=jnp.float32)
    o_ref[...] = hn
    ob_ref[...] = hn.astype(jnp.bfloat16)


def _comb_last_body(h_ref, agg_ref, cnt_ref, sew_ref, a_ref, b_ref, w_ref,
                    lb_ref, bb_ref, acc_ref):
    i = pl.program_id(0)
    h = h_ref[...]
    at = (agg_ref[0] + agg_ref[1]).astype(jnp.float32) + h
    t = jnp.dot(h, a_ref[...], preferred_element_type=jnp.float32) + lb_ref[...]
    u = jnp.dot(at, b_ref[...], preferred_element_type=jnp.float32)
    o = cnt_ref[...] * t + u + sew_ref[...] * w_ref[...] + bb_ref[...]
    rows = i * RB + lax.broadcasted_iota(jnp.int32, (RB, 1), 0)
    o = jnp.where(rows < N, o, 0.0)

    @pl.when(i == 0)
    def _():
        acc_ref[...] = jnp.zeros_like(acc_ref)

    acc_ref[...] += jnp.sum(o, axis=0, keepdims=True)

    @pl.when(i == NBLK - 1)
    def _():
        g = acc_ref[...] / jnp.float32(N)
        lane = lax.broadcasted_iota(jnp.int32, (1, D), 1)
        m = lane < NCLS
        gm = jnp.where(m, g, -1e30)
        mx = jnp.max(gm, axis=1, keepdims=True)
        ex = jnp.where(m, jnp.exp(gm - mx), 0.0)
        ls = jnp.log(jnp.sum(ex, axis=1, keepdims=True))
        acc_ref[...] = gm - mx - ls


def _pad2(a, r, c):
    return jnp.zeros((r, c), a.dtype).at[:a.shape[0], :a.shape[1]].set(a)


def kernel(x, edge_index, edge_attr, edge_flag, batch,
           W0, lin_W0, lin_b0, b0,
           W1, lin_W1, lin_b1, b1,
           W2, lin_W2, lin_b2, b2):
    e = edge_index.shape[1]
    epw = ((e + NW * 8 * CH - 1) // (NW * 8 * CH)) * 8 * CH
    epad = NW * epw
    cpw = epw // CH

    row = edge_index[0].astype(jnp.int32)
    col = edge_index[1].astype(jnp.int32)
    pad_e = epad - e
    rowp = jnp.concatenate([row, jnp.full((pad_e,), N, jnp.int32)])
    colp = jnp.concatenate([col, jnp.full((pad_e,), N, jnp.int32)])
    eap = jnp.concatenate([edge_attr, jnp.zeros((pad_e,), jnp.float32)])
    xp = jnp.zeros((NPAD, x.shape[1]), jnp.float32).at[:N].set(x)
    z1 = jnp.zeros((NPAD,), jnp.float32)
    z2 = jnp.zeros((NPAD // NS, D), jnp.bfloat16)

    degp, cntp, colr = _make_sc_pass1(epw)(rowp, colp, eap, z1)
    bl = 2048
    dinv, cnt = pl.pallas_call(
        _t1_body,
        grid=(NPAD // bl,),
        in_specs=[pl.BlockSpec((NW, bl), lambda i: (0, i)),
                  pl.BlockSpec((NW, bl), lambda i: (0, i))],
        out_specs=[pl.BlockSpec((bl, 1), lambda i: (i, 0)),
                   pl.BlockSpec((bl, 1), lambda i: (i, 0))],
        out_shape=[jax.ShapeDtypeStruct((NPAD, 1), jnp.float32),
                   jax.ShapeDtypeStruct((NPAD, 1), jnp.float32)],
    )(degp, cntp)
    dinv1 = dinv.reshape(NPAD)

    def make_sew(partp):
        return pl.pallas_call(
            _t2_body,
            grid=(NPAD // bl,),
            in_specs=[pl.BlockSpec((NW, bl), lambda i: (0, i)),
                      pl.BlockSpec((bl, 1), lambda i: (i, 0))],
            out_specs=pl.BlockSpec((bl, 1), lambda i: (i, 0)),
            out_shape=jax.ShapeDtypeStruct((NPAD, 1), jnp.float32),
        )(partp, dinv)

    rowp2 = rowp.reshape(NW * cpw, CH)
    colr2 = colr.reshape(NW * cpw, CH)
    sc_agg = _make_sc_agg(cpw)

    def prep(Wl, lWl, lbl, bbl):
        dout = Wl.shape[1]
        return (_pad2(Wl, Wl.shape[0], D),
                _pad2(lWl[:dout], D, D),
                _pad2(lWl[dout:2 * dout], D, D),
                _pad2(lWl[2 * dout][None, :], 1, D),
                _pad2(lbl[None, :], 1, D),
                _pad2(bbl[None, :], 1, D))

    params = [prep(W0, lin_W0, lin_b0, b0),
              prep(W1, lin_W1, lin_b1, b1),
              prep(W2, lin_W2, lin_b2, b2)]

    def mm(z, W):
        din = z.shape[1]
        return pl.pallas_call(
            _mm_body,
            grid=(NBLK,),
            in_specs=[pl.BlockSpec((RB, din), lambda i: (i, 0)),
                      pl.BlockSpec((din, D), lambda i: (0, 0))],
            out_specs=[pl.BlockSpec((RB, D), lambda i: (i, 0)),
                       pl.BlockSpec((RB, D), lambda i: (i, 0))],
            out_shape=[jax.ShapeDtypeStruct((NPAD, D), jnp.float32),
                       jax.ShapeDtypeStruct((NPAD, D), jnp.bfloat16)],
        )(z, W)

    def comb(h, agg, A, B, w, lb, bb, Wnext):
        last = Wnext is None
        body = _comb_last_body if last else _comb_body
        if last:
            out_spec = pl.BlockSpec((1, D), lambda i: (0, 0))
            out_shape = jax.ShapeDtypeStruct((1, D), jnp.float32)
        else:
            out_spec = [pl.BlockSpec((RB, D), lambda i: (i, 0)),
                        pl.BlockSpec((RB, D), lambda i: (i, 0))]
            out_shape = [jax.ShapeDtypeStruct((NPAD, D), jnp.float32),
                         jax.ShapeDtypeStruct((NPAD, D), jnp.bfloat16)]
        in_specs = [pl.BlockSpec((RB, D), lambda i: (i, 0)),
                    pl.BlockSpec((NC, RB, D), lambda i: (0, i, 0)),
                    pl.BlockSpec((RB, 1), lambda i: (i, 0)),
                    pl.BlockSpec((RB, 1), lambda i: (i, 0)),
                    pl.BlockSpec((D, D), lambda i: (0, 0)),
                    pl.BlockSpec((D, D), lambda i: (0, 0)),
                    pl.BlockSpec((1, D), lambda i: (0, 0)),
                    pl.BlockSpec((1, D), lambda i: (0, 0)),
                    pl.BlockSpec((1, D), lambda i: (0, 0))]
        args = [h, agg, cnt, sew, A, B, w, lb, bb]
        if not last:
            in_specs.append(pl.BlockSpec((D, D), lambda i: (0, 0)))
            args.append(Wnext)
        return pl.pallas_call(
            body,
            grid=(NBLK,),
            in_specs=in_specs,
            out_specs=out_spec,
            out_shape=out_shape,
        )(*args)

    h, hb = mm(xp, params[0][0])
    for li, (Wl, A, B, w, lb, bb) in enumerate(params):
        if li == 0:
            partp, agg = _make_sc_pass2_agg(epw)(
                rowp, colp, eap, dinv1, z1, hb, colr2, z2)
            sew = make_sew(partp)
        else:
            agg = sc_agg(hb, rowp2, colr2, z2)
        wnext = params[li + 1][0] if li < 2 else None
        h = comb(h, agg, A, B, w, lb, bb, wnext)
        if li < 2:
            h, hb = h

    return h[:, :NCLS]


# speedup vs baseline: 1.0248x; 1.0248x over previous
import functools

import jax
import jax.numpy as jnp
from jax import lax
from jax.experimental import pallas as pl
from jax.experimental.pallas import tpu as pltpu
from jax.experimental.pallas import tpu_sc as plsc

N = 10000
NPAD = 10240
NCLS = 40
D = 64
NC, NS, L = 2, 16, 16
NW = NC * NS
CH = 128
RB = 512
NBLK = NPAD // RB

_mesh = functools.partial(
    plsc.VectorSubcoreMesh, core_axis_name="c", subcore_axis_name="s",
    num_cores=NC, num_subcores=NS)

_sc_params = pltpu.CompilerParams(needs_layout_passes=False,
                                  use_tc_tiling_on_sc=False)


def _make_sc_pass1(epw):
    cpg = epw // 16

    @functools.partial(
        pl.kernel,
        out_type=(
            jax.ShapeDtypeStruct((NW, NPAD), jnp.float32),
            jax.ShapeDtypeStruct((NW, NPAD), jnp.float32),
            jax.ShapeDtypeStruct((NW * epw,), jnp.int32),
        ),
        mesh=_mesh(),
        compiler_params=_sc_params,
        scratch_types=[
            pltpu.VMEM((epw,), jnp.int32),
            pltpu.VMEM((epw,), jnp.int32),
            pltpu.VMEM((epw,), jnp.float32),
            pltpu.VMEM((epw,), jnp.int32),
            pltpu.VMEM((NPAD,), jnp.float32),
            pltpu.VMEM((NPAD,), jnp.float32),
        ],
    )
    def sc_pass1(rowh, colh, eah, z1h, degp, cntp, colrh,
                 rv, cv, av, crv, dacc, cacc):
        wid = lax.axis_index("s") * NC + lax.axis_index("c")
        base = wid * epw
        pltpu.sync_copy(rowh.at[pl.ds(base, epw)], rv)
        pltpu.sync_copy(colh.at[pl.ds(base, epw)], cv)
        pltpu.sync_copy(eah.at[pl.ds(base, epw)], av)
        pltpu.sync_copy(z1h, dacc)
        pltpu.sync_copy(z1h, cacc)

        def body(g, carry):
            sl = pl.ds(g * 16, 16)
            r = rv[sl]
            c = cv[sl]
            a = av[sl]
            k = r != c
            plsc.addupdate_scatter(dacc, [c], jnp.where(k, jnp.abs(a), 0.0))
            plsc.addupdate_scatter(cacc, [c], jnp.where(k, 1.0, 0.0))
            crv[sl] = jnp.where(k, c, N)
            return carry

        lax.fori_loop(0, cpg, body, 0)
        pltpu.sync_copy(dacc, degp.at[wid])
        pltpu.sync_copy(cacc, cntp.at[wid])
        pltpu.sync_copy(crv, colrh.at[pl.ds(base, epw)])

    return sc_pass1


def _make_sc_pass2(epw):
    cpg = epw // 16

    @functools.partial(
        pl.kernel,
        out_type=jax.ShapeDtypeStruct((NW, NPAD), jnp.float32),
        mesh=_mesh(),
        compiler_params=_sc_params,
        scratch_types=[
            pltpu.VMEM((epw,), jnp.int32),
            pltpu.VMEM((epw,), jnp.int32),
            pltpu.VMEM((epw,), jnp.float32),
            pltpu.VMEM((NPAD,), jnp.float32),
            pltpu.VMEM((NPAD,), jnp.float32),
        ],
    )
    def sc_pass2(rowh, colh, eah, dinvh, z1h, partp, rv, cv, av, dv, pacc):
        wid = lax.axis_index("s") * NC + lax.axis_index("c")
        base = wid * epw
        pltpu.sync_copy(rowh.at[pl.ds(base, epw)], rv)
        pltpu.sync_copy(colh.at[pl.ds(base, epw)], cv)
        pltpu.sync_copy(eah.at[pl.ds(base, epw)], av)
        pltpu.sync_copy(dinvh, dv)
        pltpu.sync_copy(z1h, pacc)

        def body(g, carry):
            sl = pl.ds(g * 16, 16)
            r = rv[sl]
            c = cv[sl]
            a = av[sl]
            k = r != c
            dr = plsc.load_gather(dv, [r])
            plsc.addupdate_scatter(pacc, [c],
                                   jnp.where(k, jnp.abs(a), 0.0) * dr)
            return carry

        lax.fori_loop(0, cpg, body, 0)
        pltpu.sync_copy(pacc, partp.at[wid])

    return sc_pass2


def _make_sc_agg(cpw):
    rows_per_sub = NPAD // NS

    @functools.partial(
        pl.kernel,
        out_type=jax.ShapeDtypeStruct((NC, NPAD, D), jnp.bfloat16),
        mesh=_mesh(),
        compiler_params=_sc_params,
        scratch_types=[
            pltpu.VMEM((cpw, CH), jnp.int32),
            pltpu.VMEM((cpw, CH), jnp.int32),
            pltpu.VMEM((CH, D), jnp.bfloat16),
            pltpu.VMEM((CH, D), jnp.bfloat16),
            pltpu.VMEM_SHARED((NPAD, D), jnp.bfloat16),
            pltpu.SemaphoreType.DMA,
            pltpu.SemaphoreType.DMA,
        ],
    )
    def sc_agg(hh, rowh2, colh2, z2h, aggout, rv2, cv2, buf0, buf1, acc,
               sg0, sg1):
        c = lax.axis_index("c")
        s = lax.axis_index("s")
        wid = s * NC + c
        pltpu.sync_copy(z2h, acc.at[pl.ds(s * rows_per_sub, rows_per_sub)])
        pltpu.sync_copy(rowh2.at[pl.ds(wid * cpw, cpw)], rv2)
        pltpu.sync_copy(colh2.at[pl.ds(wid * cpw, cpw)], cv2)
        plsc.subcore_barrier()

        pltpu.async_copy(hh.at[rv2.at[0]], buf0, sg0)

        def body(jj, carry):
            j0 = 2 * jj
            pltpu.async_copy(hh.at[rv2.at[j0 + 1]], buf1, sg1)
            pltpu.make_async_copy(hh.at[rv2.at[j0]], buf0, sg0).wait()
            pltpu.sync_copy(buf0, acc.at[cv2.at[j0]], add=True)

            @pl.when(j0 + 2 < cpw)
            def _():
                pltpu.async_copy(hh.at[rv2.at[j0 + 2]], buf0, sg0)

            pltpu.make_async_copy(hh.at[rv2.at[j0 + 1]], buf1, sg1).wait()
            pltpu.sync_copy(buf1, acc.at[cv2.at[j0 + 1]], add=True)
            return carry

        lax.fori_loop(0, cpw // 2, body, 0)
        plsc.subcore_barrier()
        sl = pl.ds(s * rows_per_sub, rows_per_sub)
        pltpu.sync_copy(acc.at[sl], aggout.at[c, sl])

    return sc_agg


def _t1_body(degp_ref, cntp_ref, dinv_ref, cnt_ref):
    dsum = jnp.sum(degp_ref[...], axis=0) + 1.0
    csum = jnp.sum(cntp_ref[...], axis=0) + 1.0
    dinv_ref[...] = lax.rsqrt(dsum)[:, None]
    cnt_ref[...] = csum[:, None]


def _t2_body(partp_ref, dinv_ref, sew_ref):
    psum = jnp.sum(partp_ref[...], axis=0)[:, None]
    dv = dinv_ref[...]
    sew_ref[...] = dv * (psum + dv)


def _mm_body(z_ref, w_ref, h_ref, hb_ref):
    h = jnp.dot(z_ref[...], w_ref[...], preferred_element_type=jnp.float32)
    h_ref[...] = h
    hb_ref[...] = h.astype(jnp.bfloat16)


def _comb_body(h_ref, agg_ref, cnt_ref, sew_ref, a_ref, b_ref, w_ref,
               lb_ref, bb_ref, wn_ref, o_ref, ob_ref):
    i = pl.program_id(0)
    h = h_ref[...]
    at = (agg_ref[0] + agg_ref[1]).astype(jnp.float32) + h
    t = jnp.dot(h, a_ref[...], preferred_element_type=jnp.float32) + lb_ref[...]
    u = jnp.dot(at, b_ref[...], preferred_element_type=jnp.float32)
    o = cnt_ref[...] * t + u + sew_ref[...] * w_ref[...] + bb_ref[...]
    rows = i * RB + lax.broadcasted_iota(jnp.int32, (RB, 1), 0)
    o = jnp.maximum(jnp.where(rows < N, o, 0.0), 0.0)
    hn = jnp.dot(o, wn_ref[...], preferred_element_type=jnp.float32)
    o_ref[...] = hn
    ob_ref[...] = hn.astype(jnp.bfloat16)


def _comb_last_body(h_ref, agg_ref, cnt_ref, sew_ref, a_ref, b_ref, w_ref,
                    lb_ref, bb_ref, acc_ref):
    i = pl.program_id(0)
    h = h_ref[...]
    at = (agg_ref[0] + agg_ref[1]).astype(jnp.float32) + h
    t = jnp.dot(h, a_ref[...], preferred_element_type=jnp.float32) + lb_ref[...]
    u = jnp.dot(at, b_ref[...], preferred_element_type=jnp.float32)
    o = cnt_ref[...] * t + u + sew_ref[...] * w_ref[...] + bb_ref[...]
    rows = i * RB + lax.broadcasted_iota(jnp.int32, (RB, 1), 0)
    o = jnp.where(rows < N, o, 0.0)

    @pl.when(i == 0)
    def _():
        acc_ref[...] = jnp.zeros_like(acc_ref)

    acc_ref[...] += jnp.sum(o, axis=0, keepdims=True)

    @pl.when(i == NBLK - 1)
    def _():
        g = acc_ref[...] / jnp.float32(N)
        lane = lax.broadcasted_iota(jnp.int32, (1, D), 1)
        m = lane < NCLS
        gm = jnp.where(m, g, -1e30)
        mx = jnp.max(gm, axis=1, keepdims=True)
        ex = jnp.where(m, jnp.exp(gm - mx), 0.0)
        ls = jnp.log(jnp.sum(ex, axis=1, keepdims=True))
        acc_ref[...] = gm - mx - ls


def _pad2(a, r, c):
    return jnp.zeros((r, c), a.dtype).at[:a.shape[0], :a.shape[1]].set(a)


def kernel(x, edge_index, edge_attr, edge_flag, batch,
           W0, lin_W0, lin_b0, b0,
           W1, lin_W1, lin_b1, b1,
           W2, lin_W2, lin_b2, b2):
    e = edge_index.shape[1]
    epw = ((e + NW * 8 * CH - 1) // (NW * 8 * CH)) * 8 * CH
    epad = NW * epw
    cpw = epw // CH

    row = edge_index[0].astype(jnp.int32)
    col = edge_index[1].astype(jnp.int32)
    pad_e = epad - e
    rowp = jnp.concatenate([row, jnp.full((pad_e,), N, jnp.int32)])
    colp = jnp.concatenate([col, jnp.full((pad_e,), N, jnp.int32)])
    eap = jnp.concatenate([edge_attr, jnp.zeros((pad_e,), jnp.float32)])
    xp = jnp.zeros((NPAD, x.shape[1]), jnp.float32).at[:N].set(x)
    z1 = jnp.zeros((NPAD,), jnp.float32)
    z2 = jnp.zeros((NPAD // NS, D), jnp.bfloat16)

    degp, cntp, colr = _make_sc_pass1(epw)(rowp, colp, eap, z1)
    bl = 2048
    dinv, cnt = pl.pallas_call(
        _t1_body,
        grid=(NPAD // bl,),
        in_specs=[pl.BlockSpec((NW, bl), lambda i: (0, i)),
                  pl.BlockSpec((NW, bl), lambda i: (0, i))],
        out_specs=[pl.BlockSpec((bl, 1), lambda i: (i, 0)),
                   pl.BlockSpec((bl, 1), lambda i: (i, 0))],
        out_shape=[jax.ShapeDtypeStruct((NPAD, 1), jnp.float32),
                   jax.ShapeDtypeStruct((NPAD, 1), jnp.float32)],
    )(degp, cntp)
    dinv1 = dinv.reshape(NPAD)
    partp = _make_sc_pass2(epw)(rowp, colp, eap, dinv1, z1)
    sew = pl.pallas_call(
        _t2_body,
        grid=(NPAD // bl,),
        in_specs=[pl.BlockSpec((NW, bl), lambda i: (0, i)),
                  pl.BlockSpec((bl, 1), lambda i: (i, 0))],
        out_specs=pl.BlockSpec((bl, 1), lambda i: (i, 0)),
        out_shape=jax.ShapeDtypeStruct((NPAD, 1), jnp.float32),
    )(partp, dinv)

    rowp2 = rowp.reshape(NW * cpw, CH)
    colr2 = colr.reshape(NW * cpw, CH)
    sc_agg = _make_sc_agg(cpw)

    def prep(Wl, lWl, lbl, bbl):
        dout = Wl.shape[1]
        return (_pad2(Wl, Wl.shape[0], D),
                _pad2(lWl[:dout], D, D),
                _pad2(lWl[dout:2 * dout], D, D),
                _pad2(lWl[2 * dout][None, :], 1, D),
                _pad2(lbl[None, :], 1, D),
                _pad2(bbl[None, :], 1, D))

    params = [prep(W0, lin_W0, lin_b0, b0),
              prep(W1, lin_W1, lin_b1, b1),
              prep(W2, lin_W2, lin_b2, b2)]

    def mm(z, W):
        din = z.shape[1]
        return pl.pallas_call(
            _mm_body,
            grid=(NBLK,),
            in_specs=[pl.BlockSpec((RB, din), lambda i: (i, 0)),
                      pl.BlockSpec((din, D), lambda i: (0, 0))],
            out_specs=[pl.BlockSpec((RB, D), lambda i: (i, 0)),
                       pl.BlockSpec((RB, D), lambda i: (i, 0))],
            out_shape=[jax.ShapeDtypeStruct((NPAD, D), jnp.float32),
                       jax.ShapeDtypeStruct((NPAD, D), jnp.bfloat16)],
        )(z, W)

    def comb(h, agg, A, B, w, lb, bb, Wnext):
        last = Wnext is None
        body = _comb_last_body if last else _comb_body
        if last:
            out_spec = pl.BlockSpec((1, D), lambda i: (0, 0))
            out_shape = jax.ShapeDtypeStruct((1, D), jnp.float32)
        else:
            out_spec = [pl.BlockSpec((RB, D), lambda i: (i, 0)),
                        pl.BlockSpec((RB, D), lambda i: (i, 0))]
            out_shape = [jax.ShapeDtypeStruct((NPAD, D), jnp.float32),
                         jax.ShapeDtypeStruct((NPAD, D), jnp.bfloat16)]
        in_specs = [pl.BlockSpec((RB, D), lambda i: (i, 0)),
                    pl.BlockSpec((NC, RB, D), lambda i: (0, i, 0)),
                    pl.BlockSpec((RB, 1), lambda i: (i, 0)),
                    pl.BlockSpec((RB, 1), lambda i: (i, 0)),
                    pl.BlockSpec((D, D), lambda i: (0, 0)),
                    pl.BlockSpec((D, D), lambda i: (0, 0)),
                    pl.BlockSpec((1, D), lambda i: (0, 0)),
                    pl.BlockSpec((1, D), lambda i: (0, 0)),
                    pl.BlockSpec((1, D), lambda i: (0, 0))]
        args = [h, agg, cnt, sew, A, B, w, lb, bb]
        if not last:
            in_specs.append(pl.BlockSpec((D, D), lambda i: (0, 0)))
            args.append(Wnext)
        return pl.pallas_call(
            body,
            grid=(NBLK,),
            in_specs=in_specs,
            out_specs=out_spec,
            out_shape=out_shape,
        )(*args)

    h, hb = mm(xp, params[0][0])
    for li, (Wl, A, B, w, lb, bb) in enumerate(params):
        agg = sc_agg(hb, rowp2, colr2, z2)
        wnext = params[li + 1][0] if li < 2 else None
        h = comb(h, agg, A, B, w, lb, bb, wnext)
        if li < 2:
            h, hb = h

    return h[:, :NCLS]
